# baseline (speedup 1.0000x reference)
import functools

import jax
import jax.numpy as jnp
from jax import lax
from jax.experimental import pallas as pl
from jax.experimental.pallas import tpu as pltpu
from jax.experimental.pallas import tpu_sc as plsc

_VOCAB = 32000
_B = 16
_M = 8
_MAXL = 2048
_NROWS = _B * _M
_NW = 32
_NS = 16
_RPW = _NROWS // _NW
_LANES = 16
_VSLICE = _VOCAB // _NS


def _sc_row_sums(logits, ids, lengths):
    mesh = plsc.VectorSubcoreMesh(core_axis_name="c", subcore_axis_name="s")

    @functools.partial(
        pl.kernel,
        mesh=mesh,
        compiler_params=pltpu.CompilerParams(needs_layout_passes=False),
        out_type=(
            jax.ShapeDtypeStruct((_NROWS, _LANES), jnp.float32),
            jax.ShapeDtypeStruct((_NW, 2 * _LANES), jnp.float32),
        ),
        scratch_types=[
            pltpu.VMEM((_VOCAB,), jnp.float32),
            pltpu.VMEM((_RPW, _MAXL), jnp.int32),
            pltpu.VMEM((_NROWS,), jnp.int32),
            pltpu.VMEM((_RPW, _LANES), jnp.float32),
            pltpu.VMEM((2 * _LANES,), jnp.float32),
            pltpu.VMEM_SHARED((_VOCAB,), jnp.float32),
            pltpu.SemaphoreType.DMA,
        ],
    )
    def body(logits_hbm, ids_hbm, lens_hbm, g_hbm, red_hbm,
             table_v, ids_v, lens_v, gbuf_v, red_v, table_sh, sem):
        cid = lax.axis_index("c")
        sid = lax.axis_index("s")
        wid = cid * _NS + sid
        row0 = wid * _RPW

        copies = [
            pltpu.async_copy(ids_hbm.at[pl.ds(row0, _RPW)], ids_v, sem),
            pltpu.async_copy(lens_hbm, lens_v, sem),
        ]

        @pl.when(sid == 0)
        def _stage():
            pltpu.sync_copy(logits_hbm, table_sh)

        plsc.subcore_barrier()
        pltpu.sync_copy(table_sh, table_v)
        for c in copies:
            c.wait()

        iota = lax.iota(jnp.int32, _LANES)

        unroll = 8
        chunk = unroll * _LANES
        for j in range(_RPW):
            len_splat = plsc.load_gather(
                lens_v, [jnp.full((_LANES,), row0 + j, jnp.int32)])
            lenr = jnp.maximum(len_splat[0], 1)
            nchunks = (lenr + chunk - 1) // chunk

            def step(t, acc, j=j, lenr=lenr):
                base = t * chunk
                for u in range(unroll):
                    idv = ids_v[j, pl.ds(base + u * _LANES, _LANES)]
                    g = plsc.load_gather(table_v, [idv])
                    msk = (base + u * _LANES + iota) < lenr
                    acc = acc + jnp.where(msk, g, jnp.zeros_like(g))
                return acc

            acc = lax.fori_loop(0, nchunks, step,
                                jnp.zeros((_LANES,), jnp.float32))
            gbuf_v[j] = acc

        vbase = sid * _VSLICE
        un_l = 5

        def mx_step(t, mx):
            b = vbase + t * (un_l * _LANES)
            for u in range(un_l):
                mx = jnp.maximum(mx, table_v[pl.ds(b + u * _LANES, _LANES)])
            return mx

        mx_vec = lax.fori_loop(0, _VSLICE // (un_l * _LANES), mx_step,
                               jnp.full((_LANES,), -jnp.inf, jnp.float32))
        mx_s = jnp.full((_LANES,), jnp.max(mx_vec), jnp.float32)

        def se_step(t, s):
            b = vbase + t * (un_l * _LANES)
            for u in range(un_l):
                s = s + jnp.exp(table_v[pl.ds(b + u * _LANES, _LANES)] - mx_s)
            return s

        s_vec = lax.fori_loop(0, _VSLICE // (un_l * _LANES), se_step,
                              jnp.zeros((_LANES,), jnp.float32))
        red_v[pl.ds(0, _LANES)] = mx_s
        red_v[pl.ds(_LANES, _LANES)] = jnp.full(
            (_LANES,), jnp.sum(s_vec), jnp.float32)

        pltpu.sync_copy(gbuf_v, g_hbm.at[pl.ds(row0, _RPW)])
        pltpu.sync_copy(red_v, red_hbm.at[wid])

    return body(logits, ids, lengths)


def _tc_finalize(g, red, lens_i):
    def body(g_ref, red_ref, len_ref, attn_ref, logps_ref, loss_ref):
        red = red_ref[...]
        mx_col = red[0:_NS, 0:1]
        s_col = red[0:_NS, _LANES:_LANES + 1]
        gmax = jnp.max(mx_col)
        lse = gmax + jnp.log(jnp.sum(s_col * jnp.exp(mx_col - gmax)))
        lens = jnp.maximum(len_ref[...], 1).astype(jnp.float32)
        g = jnp.sum(g_ref[...], axis=1).reshape(_B, _M)
        logps = g - lens * lse
        rowmax = jnp.max(logps, axis=1, keepdims=True)
        e = jnp.exp(logps - rowmax)
        attn = e / jnp.sum(e, axis=1, keepdims=True)
        attn_ref[...] = attn
        logps_ref[...] = logps
        loss_ref[...] = jnp.reshape(
            jnp.sum(-logps * attn / lens) / _NROWS, (1, 1))

    return pl.pallas_call(
        body,
        out_shape=(
            jax.ShapeDtypeStruct((_B, _M), jnp.float32),
            jax.ShapeDtypeStruct((_B, _M), jnp.float32),
            jax.ShapeDtypeStruct((1, 1), jnp.float32),
        ),
    )(g, red, lens_i)


@jax.jit
def _impl(logits, ids, lengths):
    g_raw, red = _sc_row_sums(logits, ids, lengths)
    lens_i = lengths.reshape(_B, _M)
    attn, logps, loss = _tc_finalize(g_raw, red, lens_i)
    return attn, logps, loss[0, 0]


def kernel(logits, ids, lengths):
    return _impl(logits, ids, lengths)


# speedup vs baseline: 1.0286x; 1.0286x over previous
import functools

import jax
import jax.numpy as jnp
from jax import lax
from jax.experimental import pallas as pl
from jax.experimental.pallas import tpu as pltpu
from jax.experimental.pallas import tpu_sc as plsc

_VOCAB = 32000
_B = 16
_M = 8
_MAXL = 2048
_NROWS = _B * _M
_NW = 32
_RPW = _NROWS // _NW
_LANES = 16


def _sc_row_sums(logits, ids, lengths):
    mesh = plsc.VectorSubcoreMesh(core_axis_name="c", subcore_axis_name="s")

    @functools.partial(
        pl.kernel,
        mesh=mesh,
        compiler_params=pltpu.CompilerParams(needs_layout_passes=False),
        out_type=jax.ShapeDtypeStruct((_NROWS, _LANES), jnp.float32),
        scratch_types=[
            pltpu.VMEM((_VOCAB,), jnp.float32),
            pltpu.VMEM((_RPW, _MAXL), jnp.int32),
            pltpu.VMEM((_NROWS,), jnp.int32),
            pltpu.VMEM((_RPW, _LANES), jnp.float32),
            pltpu.VMEM_SHARED((_VOCAB,), jnp.float32),
            pltpu.SemaphoreType.DMA,
        ],
    )
    def body(logits_hbm, ids_hbm, lens_hbm, g_hbm,
             table_v, ids_v, lens_v, gbuf_v, table_sh, sem):
        cid = lax.axis_index("c")
        sid = lax.axis_index("s")
        wid = sid * 2 + cid
        row0 = wid * _RPW

        copies = [
            pltpu.async_copy(ids_hbm.at[pl.ds(row0, _RPW)], ids_v, sem),
            pltpu.async_copy(lens_hbm, lens_v, sem),
        ]

        @pl.when(sid == 0)
        def _stage():
            pltpu.sync_copy(logits_hbm, table_sh)

        plsc.subcore_barrier()
        pltpu.sync_copy(table_sh, table_v)
        for c in copies:
            c.wait()

        iota = lax.iota(jnp.int32, _LANES)
        unroll = 8
        chunk = unroll * _LANES
        for j in range(_RPW):
            len_splat = plsc.load_gather(
                lens_v, [jnp.full((_LANES,), row0 + j, jnp.int32)])
            lenr = jnp.maximum(len_splat[0], 1)
            nchunks = (lenr + chunk - 1) // chunk

            def step(t, acc, j=j, lenr=lenr):
                base = t * chunk
                for u in range(unroll):
                    idv = ids_v[j, pl.ds(base + u * _LANES, _LANES)]
                    g = plsc.load_gather(table_v, [idv])
                    msk = (base + u * _LANES + iota) < lenr
                    acc = acc + jnp.where(msk, g, jnp.zeros_like(g))
                return acc

            acc = lax.fori_loop(0, nchunks, step,
                                jnp.zeros((_LANES,), jnp.float32))
            gbuf_v[j] = acc

        pltpu.sync_copy(gbuf_v, g_hbm.at[pl.ds(row0, _RPW)])

    return body(logits, ids, lengths)


def _tc_finalize(logits2d, g, lens_i):
    def body(lg_ref, g_ref, len_ref, attn_ref, logps_ref, loss_ref):
        x = lg_ref[...]
        mx = jnp.max(x)
        lse = mx + jnp.log(jnp.sum(jnp.exp(x - mx)))
        lens = jnp.maximum(len_ref[...], 1).astype(jnp.float32)
        g = jnp.sum(g_ref[...], axis=1).reshape(_B, _M)
        logps = g - lens * lse
        rowmax = jnp.max(logps, axis=1, keepdims=True)
        e = jnp.exp(logps - rowmax)
        attn = e / jnp.sum(e, axis=1, keepdims=True)
        attn_ref[...] = attn
        logps_ref[...] = logps
        loss_ref[...] = jnp.reshape(
            jnp.sum(-logps * attn / lens) / _NROWS, (1, 1))

    return pl.pallas_call(
        body,
        out_shape=(
            jax.ShapeDtypeStruct((_B, _M), jnp.float32),
            jax.ShapeDtypeStruct((_B, _M), jnp.float32),
            jax.ShapeDtypeStruct((1, 1), jnp.float32),
        ),
    )(logits2d, g, lens_i)


@jax.jit
def _impl(logits, ids, lengths):
    g_raw = _sc_row_sums(logits, ids, lengths)
    logits2d = logits.reshape(250, 128)
    lens_i = lengths.reshape(_B, _M)
    attn, logps, loss = _tc_finalize(logits2d, g_raw, lens_i)
    return attn, logps, loss[0, 0]


def kernel(logits, ids, lengths):
    return _impl(logits, ids, lengths)
